# baseline (speedup 1.0000x reference)
import functools

import jax
import jax.numpy as jnp
from jax import lax
from jax.experimental import pallas as pl
from jax.experimental.pallas import tpu as pltpu
from jax.experimental.pallas import tpu_sc as plsc

B = 16384
D = 32
CHUNK = 128


@functools.lru_cache(maxsize=None)
def _make_gather3():
    nc, ns = 2, 16
    nw = nc * ns
    bpw = B // nw
    nchunk = bpw // CHUNK
    mesh = plsc.VectorSubcoreMesh(core_axis_name="c", subcore_axis_name="s",
                                  num_cores=nc, num_subcores=ns)

    @functools.partial(
        pl.kernel,
        mesh=mesh,
        compiler_params=pltpu.CompilerParams(use_tc_tiling_on_sc=False),
        out_type=[jax.ShapeDtypeStruct((B, D), jnp.float32) for _ in range(3)],
        scratch_types=[
            pltpu.VMEM((nchunk, CHUNK), jnp.int32),
            pltpu.VMEM((nchunk, CHUNK), jnp.int32),
            pltpu.VMEM((nchunk, CHUNK), jnp.int32),
            pltpu.VMEM((bpw, D), jnp.float32),
            pltpu.VMEM((bpw, D), jnp.float32),
            pltpu.VMEM((bpw, D), jnp.float32),
            pltpu.SemaphoreType.DMA,
            pltpu.SemaphoreType.DMA,
            pltpu.SemaphoreType.DMA,
        ],
    )
    def gather3(uidx_hbm, iidx_hbm, sidx_hbm, ut_hbm, it_hbm, st_hbm,
                uout, iout, sout,
                uidx_v, iidx_v, sidx_v, urows, irows, srows, su, si, ss):
        wid = lax.axis_index("s") * nc + lax.axis_index("c")
        base = wid * bpw
        crow = wid * nchunk
        pltpu.sync_copy(uidx_hbm.at[pl.ds(crow, nchunk)], uidx_v)
        pltpu.sync_copy(iidx_hbm.at[pl.ds(crow, nchunk)], iidx_v)
        pltpu.sync_copy(sidx_hbm.at[pl.ds(crow, nchunk)], sidx_v)
        copies = []
        for c in range(nchunk):
            dst = pl.ds(c * CHUNK, CHUNK)
            copies.append(pltpu.async_copy(ut_hbm.at[uidx_v.at[c]], urows.at[dst], su))
            copies.append(pltpu.async_copy(it_hbm.at[iidx_v.at[c]], irows.at[dst], si))
            copies.append(pltpu.async_copy(st_hbm.at[sidx_v.at[c]], srows.at[dst], ss))
        for cp in copies:
            cp.wait()
        pltpu.sync_copy(urows, uout.at[pl.ds(base, bpw)])
        pltpu.sync_copy(irows, iout.at[pl.ds(base, bpw)])
        pltpu.sync_copy(srows, sout.at[pl.ds(base, bpw)])

    return gather3


def _mlp_body(u_ref, i_ref, s_ref, w1_ref, b1_ref, w2_ref, b2_ref, w3_ref,
              b3_ref, o_ref):
    h = (jnp.dot(u_ref[...], w1_ref[0:D, :], preferred_element_type=jnp.float32)
         + jnp.dot(i_ref[...], w1_ref[D:2 * D, :], preferred_element_type=jnp.float32)
         + jnp.dot(s_ref[...], w1_ref[2 * D:3 * D, :], preferred_element_type=jnp.float32)
         + b1_ref[...])
    h = jnp.maximum(h, 0.0)
    h = jnp.dot(h, w2_ref[...], preferred_element_type=jnp.float32) + b2_ref[...]
    h = jnp.maximum(h, 0.0)
    o = jnp.dot(h, w3_ref[...], preferred_element_type=jnp.float32) + b3_ref[...]
    o_ref[...] = 1.0 / (1.0 + jnp.exp(-o))


def _mlp(u_emb, i_emb, s_emb, W1, b1, W2, b2, W3, b3, bm=2048):
    grid = (B // bm,)
    return pl.pallas_call(
        _mlp_body,
        grid=grid,
        in_specs=[
            pl.BlockSpec((bm, D), lambda i: (i, 0)),
            pl.BlockSpec((bm, D), lambda i: (i, 0)),
            pl.BlockSpec((bm, D), lambda i: (i, 0)),
            pl.BlockSpec((3 * D, 256), lambda i: (0, 0)),
            pl.BlockSpec((1, 256), lambda i: (0, 0)),
            pl.BlockSpec((256, 128), lambda i: (0, 0)),
            pl.BlockSpec((1, 128), lambda i: (0, 0)),
            pl.BlockSpec((128, 1), lambda i: (0, 0)),
            pl.BlockSpec((1, 1), lambda i: (0, 0)),
        ],
        out_specs=pl.BlockSpec((bm, 1), lambda i: (i, 0)),
        out_shape=jax.ShapeDtypeStruct((B, 1), jnp.float32),
    )(u_emb, i_emb, s_emb, W1, b1, W2, b2, W3, b3)


def kernel(user_input, pos_item_input, pos_item_subcategory_input,
           user_table, item_table, sub_table,
           W1, b1, W2, b2, W3, b3):
    uidx = user_input.astype(jnp.int32).reshape(B // CHUNK, CHUNK)
    iidx = pos_item_input.astype(jnp.int32).reshape(B // CHUNK, CHUNK)
    sidx = pos_item_subcategory_input.astype(jnp.int32).reshape(B // CHUNK, CHUNK)
    u_emb, i_emb, s_emb = _make_gather3()(uidx, iidx, sidx,
                                          user_table, item_table, sub_table)
    return _mlp(u_emb, i_emb, s_emb, W1,
                b1.reshape(1, 256), W2, b2.reshape(1, 128),
                W3, b3.reshape(1, 1))
